# baseline (speedup 1.0000x reference)
import jax
import jax.numpy as jnp
import numpy as np
from jax import lax
from jax.experimental import pallas as pl
from jax.experimental.pallas import tpu as pltpu

N_NODES = 10000
N_EDGES = 160000
H = 8

NPAD = 10240
EPAD = 163840
DUMMY = 10000
RB = NPAD // 8

_SELU_SCALE = 1.0507009873554805
_SELU_ALPHA = 1.6732632423543772


def _selu(t):
    neg = _SELU_ALPHA * (jnp.exp(jnp.minimum(t, 0.0)) - 1.0)
    return _SELU_SCALE * jnp.where(t > 0, t, neg)


_P66 = 66 * 66
_PFLAT = 4544


def _cnn_body(cf_ref, c1w_ref, c1b_ref, c2w_ref, c2b_ref, mask_ref, out_ref,
              xp1, xp2):
    xp1[...] = jnp.zeros((4, _PFLAT), jnp.float32)
    xp2[...] = jnp.zeros((32, _PFLAT), jnp.float32)
    for r in range(64):
        xp1[:, (r + 1) * 66 + 1:(r + 1) * 66 + 65] = cf_ref[:, r * 64:(r + 1) * 64]
    y1 = jnp.zeros((32, _P66), jnp.float32)
    for dy in range(3):
        for dx in range(3):
            k = dy * 66 + dx
            xs = xp1[:, k:k + _P66]
            y1 = y1 + jnp.dot(c1w_ref[:, :, dy, dx], xs,
                              preferred_element_type=jnp.float32)
    m = mask_ref[:, 67:67 + _P66]
    y1 = jnp.maximum(y1 + c1b_ref[...], 0.0) * m
    xp2[:, 67:67 + _P66] = y1
    y2 = jnp.zeros((16, _P66), jnp.float32)
    for dy in range(3):
        for dx in range(3):
            k = dy * 66 + dx
            xs = xp2[:, k:k + _P66]
            y2 = y2 + jnp.dot(c2w_ref[:, :, dy, dx], xs,
                              preferred_element_type=jnp.float32)
    y2 = jnp.maximum(y2 + c2b_ref[...], 0.0) * m
    out_ref[...] = (jnp.sum(y2, axis=1) * (1.0 / 4096.0))[None, :]


def _cnn_call(cf, c1w, c1b, c2w, c2b):
    mask_np = np.zeros((1, _PFLAT), np.float32)
    for r in range(1, 65):
        mask_np[0, r * 66 + 1:r * 66 + 65] = 1.0
    return pl.pallas_call(
        _cnn_body,
        out_shape=jax.ShapeDtypeStruct((1, 16), jnp.float32),
        scratch_shapes=[pltpu.VMEM((4, _PFLAT), jnp.float32),
                        pltpu.VMEM((32, _PFLAT), jnp.float32)],
    )(cf.reshape(4, 4096), c1w, c1b.reshape(32, 1), c2w, c2b.reshape(16, 1),
      jnp.asarray(mask_np))


def _prep_body(h_ref, w_ref, as_ref, ad_ref, hp_ref, es_ref, ed_ref,
               esm_ref, edm_ref):
    i = pl.program_id(0)
    hf = as_ref.shape[0]
    hp = jnp.dot(h_ref[...], w_ref[...], preferred_element_type=jnp.float32)
    es = jnp.dot(hp, as_ref[...], preferred_element_type=jnp.float32)
    ed = jnp.dot(hp, ad_ref[...], preferred_element_type=jnp.float32)
    hp_ref[:, :hf] = hp
    hp_ref[:, hf:] = es
    es_ref[...] = es
    ed_ref[...] = ed
    bs = jnp.max(es, axis=0)[None, :]
    bd = jnp.max(ed, axis=0)[None, :]

    @pl.when(i == 0)
    def _():
        esm_ref[...] = bs
        edm_ref[...] = bd

    @pl.when(i != 0)
    def _():
        esm_ref[...] = jnp.maximum(esm_ref[...], bs)
        edm_ref[...] = jnp.maximum(edm_ref[...], bd)


def _prep_call(h, wp, asb, adb):
    fip = h.shape[1]
    hf = wp.shape[1]
    return pl.pallas_call(
        _prep_body,
        grid=(8,),
        in_specs=[
            pl.BlockSpec((RB, fip), lambda i: (i, 0)),
            pl.BlockSpec((fip, hf), lambda i: (0, 0)),
            pl.BlockSpec((hf, 128), lambda i: (0, 0)),
            pl.BlockSpec((hf, 128), lambda i: (0, 0)),
        ],
        out_specs=[
            pl.BlockSpec((RB, hf + 128), lambda i: (i, 0)),
            pl.BlockSpec((RB, 128), lambda i: (i, 0)),
            pl.BlockSpec((RB, 128), lambda i: (i, 0)),
            pl.BlockSpec((1, 128), lambda i: (0, 0)),
            pl.BlockSpec((1, 128), lambda i: (0, 0)),
        ],
        out_shape=[
            jax.ShapeDtypeStruct((NPAD, hf + 128), jnp.float32),
            jax.ShapeDtypeStruct((NPAD, 128), jnp.float32),
            jax.ShapeDtypeStruct((NPAD, 128), jnp.float32),
            jax.ShapeDtypeStruct((1, 128), jnp.float32),
            jax.ShapeDtypeStruct((1, 128), jnp.float32),
        ],
    )(h, wp, asb, adb)


def _fin_body(a0_ref, a1_ref, b_ref, out_ref):
    out_ref[...] = _selu(a0_ref[...] + a1_ref[...] + b_ref[...])


def _fin4_body(a0_ref, a1_ref, b_ref, xc_ref, out_ref):
    res = _selu(a0_ref[...] + a1_ref[...] + b_ref[...])
    xc = xc_ref[...]
    col = lax.broadcasted_iota(jnp.int32, res.shape, 1)
    is0 = col == 0
    is1 = col == 1
    res = res + jnp.where(is0 | is1, xc, 0.0)
    up = (xc == 1.0) & is0
    dn = (xc == 0.0) & is0
    lf = (xc == 0.0) & is1
    rt = (xc == 1.0) & is1
    res = jnp.where(up, 1.0, jnp.where(dn, 0.0, res))
    res = jnp.where(lf, 0.0, jnp.where(rt, 1.0, res))
    out_ref[...] = res


def _fin_call(a0, a1, bp):
    fop = a0.shape[1]
    return pl.pallas_call(
        _fin_body,
        grid=(8,),
        in_specs=[
            pl.BlockSpec((RB, fop), lambda i: (i, 0)),
            pl.BlockSpec((RB, fop), lambda i: (i, 0)),
            pl.BlockSpec((1, fop), lambda i: (0, 0)),
        ],
        out_specs=pl.BlockSpec((RB, fop), lambda i: (i, 0)),
        out_shape=jax.ShapeDtypeStruct((NPAD, fop), jnp.float32),
    )(a0, a1, bp)


def _fin4_call(a0, a1, bp, xc):
    fop = a0.shape[1]
    return pl.pallas_call(
        _fin4_body,
        grid=(8,),
        in_specs=[
            pl.BlockSpec((RB, fop), lambda i: (i, 0)),
            pl.BlockSpec((RB, fop), lambda i: (i, 0)),
            pl.BlockSpec((1, fop), lambda i: (0, 0)),
            pl.BlockSpec((RB, fop), lambda i: (i, 0)),
        ],
        out_specs=pl.BlockSpec((RB, fop), lambda i: (i, 0)),
        out_shape=jax.ShapeDtypeStruct((NPAD, fop), jnp.float32),
    )(a0, a1, bp, xc)


_DIMS = [(26, 32), (32, 64), (64, 32), (32, 26)]
_PADS = [(32, 32), (32, 64), (64, 32), (32, 32)]


def _pad_weights(W, a_s, a_d, b, fi, fo, fip, fop):
    Wr = W.reshape(fi, H, fo)
    Wp = jnp.zeros((fip, H, fop), jnp.float32).at[:fi, :, :fo].set(Wr)
    Wp = Wp.reshape(fip, H * fop)
    rows = (np.arange(H)[:, None] * fop + np.arange(fo)[None, :]).ravel()
    cols = np.repeat(np.arange(H), fo)
    asb = jnp.zeros((H * fop, 128), jnp.float32).at[rows, cols].set(a_s.ravel())
    adb = jnp.zeros((H * fop, 128), jnp.float32).at[rows, cols].set(a_d.ravel())
    bp = jnp.zeros((1, fop), jnp.float32).at[0, :fo].set(b)
    return Wp, asb, adb, bp


def kernel(x, conv_feat, c1w, c1b, c2w, c2b, W1, as1, ad1, b1, W2, as2, ad2,
           b2, W3, as3, ad3, b3, W4, as4, ad4, b4, edge_index):
    src = edge_index[0].astype(jnp.int32)
    dst = edge_index[1].astype(jnp.int32)
    srcp = jnp.full((EPAD,), DUMMY, jnp.int32).at[:N_EDGES].set(src)
    dstp = jnp.full((EPAD,), DUMMY, jnp.int32).at[:N_EDGES].set(dst)

    feat = _cnn_call(conv_feat[0], c1w, c1b, c2w, c2b)

    params = []
    for (fi, fo), (fip, fop), (W, a_s, a_d, b) in zip(
            _DIMS, _PADS,
            [(W1, as1, ad1, b1), (W2, as2, ad2, b2),
             (W3, as3, ad3, b3), (W4, as4, ad4, b4)]):
        params.append(_pad_weights(W, a_s, a_d, b, fi, fo, fip, fop))

    h = jnp.zeros((NPAD, 32), jnp.float32)
    h = h.at[:N_NODES, :16].set(jnp.broadcast_to(feat, (N_NODES, 16)))
    h = h.at[:N_NODES, 16:26].set(x)
    xc = jnp.zeros((NPAD, 32), jnp.float32)
    xc = xc.at[:N_NODES, 0].set(x[:, 0]).at[:N_NODES, 1].set(x[:, 1])

    for _ in range(3):
        for li in range(4):
            Wp, asb, adb, bp = params[li]
            hpx, es, ed, esm, edm = _prep_call(h, Wp, asb, adb)
            cval = jnp.max(esm[0, :8]) + jnp.max(edm[0, :8])
            fop = _PADS[li][1]
            hf = H * fop
            e = es[:, :H][srcp] + ed[:, :H][dstp]
            e = jnp.maximum(e, 0.2 * e)
            p = jnp.exp(e - cval)
            den = jax.ops.segment_sum(p, dstp, num_segments=NPAD)
            w = p / (H * den[dstp] + 1e-30)
            hsrc = hpx[:, :hf][srcp].reshape(EPAD, H, fop)
            msg = jnp.einsum('eh,ehk->ek', w, hsrc)
            agg0 = jax.ops.segment_sum(msg, dstp, num_segments=NPAD)
            agg1 = jnp.zeros((NPAD, fop), jnp.float32)
            if li < 3:
                h = _fin_call(agg0, agg1, bp)
            else:
                h = _fin4_call(agg0, agg1, bp, xc)
    return h[:N_NODES, :2]
